# Initial kernel scaffold; baseline (speedup 1.0000x reference)
import jax
import jax.numpy as jnp
from jax.experimental import pallas as pl


def kernel(x, edge_index, edge_attr, u, batch, W1, b1, W2, b2, W3, b3, W4, b4, W5, b5):
    raise NotImplementedError("write your pallas kernel here")


# speedup vs baseline: 1.5908x; 1.5908x over previous
import functools

import jax
import jax.numpy as jnp
from jax import lax
from jax.experimental import pallas as pl
from jax.experimental.pallas import tpu as pltpu
from jax.experimental.pallas import tpu_sc as plsc

NC, NS = 2, 16
NW = NC * NS
CH = 128
BE = 4000
BN = 10000


def _sc_gather(x, src):
    N, D = x.shape
    E = src.shape[0]
    n_chunks = E // CH
    assert n_chunks * CH == E
    per_worker = -(-n_chunks // NW)

    mesh = plsc.VectorSubcoreMesh(
        core_axis_name="c", subcore_axis_name="s", num_cores=NC, num_subcores=NS)

    @functools.partial(
        pl.kernel,
        out_type=jax.ShapeDtypeStruct((E, D), jnp.float32),
        mesh=mesh,
        scratch_types=[
            pltpu.VMEM((CH,), jnp.int32),
            pltpu.VMEM((CH, D), jnp.float32),
            pltpu.SemaphoreType.DMA,
        ],
        compiler_params=pltpu.CompilerParams(use_tc_tiling_on_sc=False),
    )
    def gather_k(x_hbm, src_hbm, out_hbm, idx_v, rows_v, sem):
        wid = lax.axis_index("s") * NC + lax.axis_index("c")

        def body(i, _):
            cid = wid * per_worker + i

            @pl.when(cid < n_chunks)
            def _():
                base = cid * CH
                pltpu.sync_copy(src_hbm.at[pl.ds(base, CH)], idx_v)
                pltpu.async_copy(x_hbm.at[idx_v], rows_v, sem).wait()
                pltpu.sync_copy(rows_v, out_hbm.at[pl.ds(base, CH)])
            return ()

        lax.fori_loop(0, per_worker, body, (), unroll=False)

    return gather_k(x, src)


def _tc_edge_mlp(xs, ea, W1, b1, W2, b2):
    E = xs.shape[0]
    grid = E // BE
    assert grid * BE == E

    def body(xs_ref, ea_ref, W1_ref, b1_ref, W2_ref, b2_ref, out_ref):
        xs_b = xs_ref[...]
        z = (xs_b[:, 0:1] * W1_ref[0:1, :]
             + xs_b[:, 1:2] * W1_ref[1:2, :]
             + ea_ref[...] * W1_ref[2:3, :]
             + b1_ref[...])
        h1 = jnp.maximum(z, 0.0)
        h2 = jnp.dot(h1, W2_ref[...], preferred_element_type=jnp.float32)
        h2 = jnp.maximum(h2 + b2_ref[...], 0.0)
        colid = lax.broadcasted_iota(jnp.int32, (BE, 16), 1)
        tail = jnp.where(colid == 0, 1.0, 0.0).astype(jnp.float32)
        out_ref[...] = jnp.concatenate([h2, tail], axis=1)

    return pl.pallas_call(
        body,
        grid=(grid,),
        in_specs=[
            pl.BlockSpec((BE, 16), lambda i: (i, 0)),
            pl.BlockSpec((BE, 1), lambda i: (i, 0)),
            pl.BlockSpec((3, 64), lambda i: (0, 0)),
            pl.BlockSpec((1, 64), lambda i: (0, 0)),
            pl.BlockSpec((64, 64), lambda i: (0, 0)),
            pl.BlockSpec((1, 64), lambda i: (0, 0)),
        ],
        out_specs=pl.BlockSpec((BE, 80), lambda i: (i, 0)),
        out_shape=jax.ShapeDtypeStruct((E, 80), jnp.float32),
    )(xs, ea, W1, b1, W2, b2)


def _sc_scatter(h2x, dest, zeros16):
    E, W = h2x.shape
    N = zeros16.shape[0]
    n_stripes = W // 16
    assert n_stripes * 16 == W
    n_chunks = E // CH
    assert n_chunks * CH == E
    per_tile = -(-n_chunks // NS)
    rows_per_tile = N // NS
    assert rows_per_tile * NS == N
    passes = -(-n_stripes // NC)

    mesh = plsc.VectorSubcoreMesh(
        core_axis_name="c", subcore_axis_name="s", num_cores=NC, num_subcores=NS)

    @functools.partial(
        pl.kernel,
        out_type=jax.ShapeDtypeStruct((N, W), jnp.float32),
        mesh=mesh,
        scratch_types=[
            pltpu.VMEM((CH,), jnp.int32),
            pltpu.VMEM((CH, 16), jnp.float32),
            pltpu.VMEM_SHARED((N, 16), jnp.float32),
        ],
        compiler_params=pltpu.CompilerParams(use_tc_tiling_on_sc=False),
    )
    def scatter_k(h2_hbm, dest_hbm, zeros_hbm, out_hbm, idx_v, dat_v, acc_sh):
        cid = lax.axis_index("c")
        sid = lax.axis_index("s")
        row0 = sid * rows_per_tile

        for p in range(passes):
            stripe = p * NC + cid

            @pl.when(stripe < n_stripes)
            def _():
                pltpu.sync_copy(zeros_hbm.at[pl.ds(row0, rows_per_tile)],
                                acc_sh.at[pl.ds(row0, rows_per_tile)])

            plsc.subcore_barrier()

            @pl.when(stripe < n_stripes)
            def _():
                col0 = stripe * 16

                def body(i, _):
                    ck = i * NS + sid

                    @pl.when(ck < n_chunks)
                    def _():
                        base = ck * CH
                        pltpu.sync_copy(dest_hbm.at[pl.ds(base, CH)], idx_v)
                        pltpu.sync_copy(
                            h2_hbm.at[pl.ds(base, CH), pl.ds(col0, 16)], dat_v)
                        pltpu.sync_copy(dat_v, acc_sh.at[idx_v], add=True)
                    return ()

                lax.fori_loop(0, per_tile, body, (), unroll=False)

            plsc.subcore_barrier()

            @pl.when(stripe < n_stripes)
            def _():
                col0 = stripe * 16
                pltpu.sync_copy(
                    acc_sh.at[pl.ds(row0, rows_per_tile)],
                    out_hbm.at[pl.ds(row0, rows_per_tile), pl.ds(col0, 16)])

            plsc.subcore_barrier()

    return scatter_k(h2x, dest, zeros16)


def _tc_node_mlp(x, s80, batch2d, u2d, W3, b3, W4x, W4m, w4u, b4, W5, b5):
    N = x.shape[0]
    grid = N // BN
    assert grid * BN == N
    G = u2d.shape[0]

    def body(x_ref, s_ref, b_ref, u_ref, W3_ref, b3_ref, W4x_ref, W4m_ref,
             w4u_ref, b4_ref, W5_ref, b5_ref, out_ref):
        s = s_ref[...]
        sums = s[:, 0:64]
        c = s[:, 64:65]
        mean = sums / jnp.maximum(c, 1.0)
        m3 = jnp.dot(mean, W3_ref[...], preferred_element_type=jnp.float32)
        m3 = jnp.where(c > 0.5, m3 + b3_ref[...], 0.0)
        bb = b_ref[...]
        oh = (lax.broadcasted_iota(jnp.int32, (BN, G), 1) == bb)
        ub = jnp.dot(oh.astype(jnp.float32), u_ref[...],
                     preferred_element_type=jnp.float32)
        xb = x_ref[...]
        g = (xb[:, 0:1] * W4x_ref[0:1, :]
             + xb[:, 1:2] * W4x_ref[1:2, :]
             + jnp.dot(m3, W4m_ref[...], preferred_element_type=jnp.float32)
             + ub * w4u_ref[...]
             + b4_ref[...])
        h = jnp.maximum(g, 0.0)
        out = jnp.dot(h, W5_ref[...], preferred_element_type=jnp.float32)
        out_ref[...] = out + b5_ref[...]

    return pl.pallas_call(
        body,
        grid=(grid,),
        in_specs=[
            pl.BlockSpec((BN, 2), lambda i: (i, 0)),
            pl.BlockSpec((BN, 80), lambda i: (i, 0)),
            pl.BlockSpec((BN, 1), lambda i: (i, 0)),
            pl.BlockSpec((G, 1), lambda i: (0, 0)),
            pl.BlockSpec((64, 64), lambda i: (0, 0)),
            pl.BlockSpec((1, 64), lambda i: (0, 0)),
            pl.BlockSpec((2, 67), lambda i: (0, 0)),
            pl.BlockSpec((64, 67), lambda i: (0, 0)),
            pl.BlockSpec((1, 67), lambda i: (0, 0)),
            pl.BlockSpec((1, 67), lambda i: (0, 0)),
            pl.BlockSpec((67, 2), lambda i: (0, 0)),
            pl.BlockSpec((1, 2), lambda i: (0, 0)),
        ],
        out_specs=pl.BlockSpec((BN, 2), lambda i: (i, 0)),
        out_shape=jax.ShapeDtypeStruct((N, 2), jnp.float32),
    )(x, s80, batch2d, u2d, W3, b3, W4x, W4m, w4u, b4, W5, b5)


def kernel(x, edge_index, edge_attr, u, batch, W1, b1, W2, b2, W3, b3, W4, b4, W5, b5):
    N = x.shape[0]
    src = edge_index[0]
    dest = edge_index[1]

    x16 = jnp.pad(x, ((0, 0), (0, 14)))
    xs = _sc_gather(x16, src)
    h2x = _tc_edge_mlp(xs, edge_attr, W1, b1.reshape(1, -1), W2,
                       b2.reshape(1, -1))
    s80 = _sc_scatter(h2x, dest, jnp.zeros((N, 16), jnp.float32))
    out = _tc_node_mlp(
        x, s80, batch.reshape(-1, 1), u.reshape(-1, 1),
        W3, b3.reshape(1, -1),
        W4[0:2, :], W4[2:66, :], W4[66:67, :], b4.reshape(1, -1),
        W5, b5.reshape(1, -1))
    return out


# speedup vs baseline: 2.0625x; 1.2965x over previous
import functools

import jax
import jax.numpy as jnp
from jax import lax
from jax.experimental import pallas as pl
from jax.experimental.pallas import tpu as pltpu
from jax.experimental.pallas import tpu_sc as plsc

NC, NS = 2, 16
NW = NC * NS
CH = 128
BE = 4000
BN = 4000


KB = 4


def _sc_gather(x, src, dest):
    N, D = x.shape
    E = src.shape[0]
    n_chunks = E // CH
    assert n_chunks * CH == E
    src = src.reshape(n_chunks, CH)
    dest = dest.reshape(n_chunks, CH)
    per_worker = -(-n_chunks // NW)
    n_super = -(-per_worker // KB)
    SCH = KB * CH
    RED = 2000
    assert N % RED == 0 and RED % 16 == 0
    n_pieces = N // RED
    pieces_per_tile = -(-n_pieces // NS)

    mesh = plsc.VectorSubcoreMesh(
        core_axis_name="c", subcore_axis_name="s", num_cores=NC, num_subcores=NS)

    @functools.partial(
        pl.kernel,
        out_type=[jax.ShapeDtypeStruct((E, D), jnp.float32),
                  jax.ShapeDtypeStruct((NC, N), jnp.float32),
                  jax.ShapeDtypeStruct((NC, NS, N), jnp.float32)],
        mesh=mesh,
        scratch_types=[
            pltpu.VMEM((KB, CH), jnp.int32),
            pltpu.VMEM((KB, CH), jnp.int32),
            pltpu.VMEM((SCH, 16), jnp.float32),
            pltpu.VMEM((N,), jnp.float32),
            pltpu.VMEM((RED,), jnp.float32),
            pltpu.VMEM((RED,), jnp.float32),
            pltpu.SemaphoreType.DMA,
        ],
        compiler_params=pltpu.CompilerParams(use_tc_tiling_on_sc=False, needs_layout_passes=False),
    )
    def gather_k(x_hbm, src_hbm, dest_hbm, out_hbm, cnt_hbm, part_hbm,
                 sidx_v, didx_v, rows_v, hist_v, tmp_v, red_v, sem_g):
        cid = lax.axis_index("c")
        sid = lax.axis_index("s")
        wid = sid * NC + cid
        ones = jnp.ones((16,), jnp.float32)
        zeros = jnp.zeros((16,), jnp.float32)

        def zbody(i, _):
            hist_v[pl.ds(i * 16, 16)] = zeros
            return ()
        lax.fori_loop(0, N // 16, zbody, (), unroll=False)

        def sbody(g, _):
            c0 = wid * per_worker + g * KB
            base = c0 * CH
            nch = jnp.maximum(jnp.minimum(
                jnp.minimum(per_worker - g * KB, n_chunks - c0), KB), 0)

            @pl.when(nch == KB)
            def _():
                pltpu.sync_copy(src_hbm.at[pl.ds(c0, KB)], sidx_v)
                pltpu.sync_copy(dest_hbm.at[pl.ds(c0, KB)], didx_v)

            @pl.when((nch > 0) & (nch < KB))
            def _():
                for j in range(KB):
                    @pl.when(j < nch)
                    def _():
                        pltpu.sync_copy(src_hbm.at[pl.ds(c0 + j, 1)],
                                        sidx_v.at[pl.ds(j, 1)])
                        pltpu.sync_copy(dest_hbm.at[pl.ds(c0 + j, 1)],
                                        didx_v.at[pl.ds(j, 1)])

            for j in range(KB):
                @pl.when(j < nch)
                def _():
                    pltpu.async_copy(
                        x_hbm.at[sidx_v.at[j]],
                        rows_v.at[pl.ds(j * CH, CH)], sem_g)

            for j in range(KB):
                @pl.when(j < nch)
                def _():
                    def hbody(k, _):
                        d = didx_v[j, pl.ds(k * 16, 16)]
                        plsc.addupdate_scatter(hist_v, [d], ones)
                        return ()
                    lax.fori_loop(0, CH // 16, hbody, (), unroll=False)

            for j in range(KB):
                @pl.when(j < nch)
                def _():
                    pltpu.make_async_copy(
                        x_hbm.at[sidx_v.at[j]],
                        rows_v.at[pl.ds(j * CH, CH)], sem_g).wait()

            @pl.when(nch == KB)
            def _():
                pltpu.sync_copy(rows_v, out_hbm.at[pl.ds(base, SCH)])

            @pl.when((nch > 0) & (nch < KB))
            def _():
                for j in range(KB):
                    @pl.when(j < nch)
                    def _():
                        pltpu.sync_copy(
                            rows_v.at[pl.ds(j * CH, CH)],
                            out_hbm.at[pl.ds(base + j * CH, CH)])
            return ()

        lax.fori_loop(0, n_super, sbody, (), unroll=False)

        pltpu.sync_copy(hist_v, part_hbm.at[cid, sid])
        plsc.subcore_barrier()
        for pp in range(pieces_per_tile):
            piece = pp * NS + sid

            @pl.when(piece < n_pieces)
            def _():
                p0 = piece * RED

                def rzero(i, _):
                    red_v[pl.ds(i * 16, 16)] = zeros
                    return ()
                lax.fori_loop(0, RED // 16, rzero, (), unroll=False)

                for q in range(NS):
                    pltpu.sync_copy(part_hbm.at[cid, q, pl.ds(p0, RED)], tmp_v)

                    def racc(i, _):
                        red_v[pl.ds(i * 16, 16)] = (red_v[pl.ds(i * 16, 16)]
                                                    + tmp_v[pl.ds(i * 16, 16)])
                        return ()
                    lax.fori_loop(0, RED // 16, racc, (), unroll=False)
                pltpu.sync_copy(red_v, cnt_hbm.at[cid, pl.ds(p0, RED)])

    return gather_k(x, src, dest)


def _tc_edge_mlp(xs, ea, W1, b1, W2, b2):
    E = xs.shape[0]
    grid = E // BE
    assert grid * BE == E

    def body(xs_ref, ea_ref, W1_ref, b1_ref, W2_ref, b2_ref, out_ref):
        xs_b = xs_ref[...]
        z = (xs_b[:, 0:1] * W1_ref[0:1, :]
             + xs_b[:, 1:2] * W1_ref[1:2, :]
             + ea_ref[...] * W1_ref[2:3, :]
             + b1_ref[...])
        h1 = jnp.maximum(z, 0.0)
        h2 = jnp.dot(h1, W2_ref[...], preferred_element_type=jnp.float32)
        out_ref[...] = jnp.maximum(h2 + b2_ref[...], 0.0)

    return pl.pallas_call(
        body,
        grid=(grid,),
        in_specs=[
            pl.BlockSpec((BE, 16), lambda i: (i, 0)),
            pl.BlockSpec((BE, 1), lambda i: (i, 0)),
            pl.BlockSpec((3, 64), lambda i: (0, 0)),
            pl.BlockSpec((1, 64), lambda i: (0, 0)),
            pl.BlockSpec((64, 64), lambda i: (0, 0)),
            pl.BlockSpec((1, 64), lambda i: (0, 0)),
        ],
        out_specs=pl.BlockSpec((BE, 64), lambda i: (i, 0)),
        out_shape=jax.ShapeDtypeStruct((E, 64), jnp.float32),
    )(xs, ea, W1, b1, W2, b2)


def _sc_scatter(h2x, dest, zeros16):
    E, W = h2x.shape
    N = zeros16.shape[0]
    n_stripes = W // 16
    assert n_stripes * 16 == W
    n_chunks = E // CH
    assert n_chunks * CH == E
    per_tile = -(-n_chunks // NS)
    rows_per_tile = N // NS
    assert rows_per_tile * NS == N
    passes = -(-n_stripes // NC)

    mesh = plsc.VectorSubcoreMesh(
        core_axis_name="c", subcore_axis_name="s", num_cores=NC, num_subcores=NS)

    @functools.partial(
        pl.kernel,
        out_type=jax.ShapeDtypeStruct((N, W), jnp.float32),
        mesh=mesh,
        scratch_types=[
            pltpu.VMEM((CH,), jnp.int32),
            pltpu.VMEM((CH, 16), jnp.float32),
            pltpu.VMEM_SHARED((N, 16), jnp.float32),
        ],
        compiler_params=pltpu.CompilerParams(use_tc_tiling_on_sc=False, needs_layout_passes=False),
    )
    def scatter_k(h2_hbm, dest_hbm, zeros_hbm, out_hbm, idx_v, dat_v, acc_sh):
        cid = lax.axis_index("c")
        sid = lax.axis_index("s")
        row0 = sid * rows_per_tile

        for p in range(passes):
            stripe = p * NC + cid

            @pl.when(stripe < n_stripes)
            def _():
                pltpu.sync_copy(zeros_hbm.at[pl.ds(row0, rows_per_tile)],
                                acc_sh.at[pl.ds(row0, rows_per_tile)])

            plsc.subcore_barrier()

            @pl.when(stripe < n_stripes)
            def _():
                col0 = stripe * 16

                def body(i, _):
                    ck = i * NS + sid

                    @pl.when(ck < n_chunks)
                    def _():
                        base = ck * CH
                        pltpu.sync_copy(dest_hbm.at[pl.ds(base, CH)], idx_v)
                        pltpu.sync_copy(
                            h2_hbm.at[pl.ds(base, CH), pl.ds(col0, 16)], dat_v)
                        pltpu.sync_copy(dat_v, acc_sh.at[idx_v], add=True)
                    return ()

                lax.fori_loop(0, per_tile, body, (), unroll=False)

            plsc.subcore_barrier()

            @pl.when(stripe < n_stripes)
            def _():
                col0 = stripe * 16
                pltpu.sync_copy(
                    acc_sh.at[pl.ds(row0, rows_per_tile)],
                    out_hbm.at[pl.ds(row0, rows_per_tile), pl.ds(col0, 16)])

            plsc.subcore_barrier()

    return scatter_k(h2x, dest, zeros16)


def _tc_node_mlp(x, s80, cnt_a, cnt_b, batch2d, u2d, W3, b3, W4x, W4m, w4u,
                 b4, W5, b5):
    N = x.shape[0]
    grid = N // BN
    assert grid * BN == N
    G = u2d.shape[0]

    def body(x_ref, s_ref, ca_ref, cb_ref, b_ref, u_ref, W3_ref, b3_ref,
             W4x_ref, W4m_ref, w4u_ref, b4_ref, W5_ref, b5_ref, out_ref):
        sums = s_ref[...]
        c = ca_ref[...] + cb_ref[...]
        mean = sums / jnp.maximum(c, 1.0)
        m3 = jnp.dot(mean, W3_ref[...], preferred_element_type=jnp.float32)
        m3 = jnp.where(c > 0.5, m3 + b3_ref[...], 0.0)
        bb = b_ref[...]
        oh = (lax.broadcasted_iota(jnp.int32, (BN, G), 1) == bb)
        ub = jnp.dot(oh.astype(jnp.float32), u_ref[...],
                     preferred_element_type=jnp.float32)
        xb = x_ref[...]
        g = (xb[:, 0:1] * W4x_ref[0:1, :]
             + xb[:, 1:2] * W4x_ref[1:2, :]
             + jnp.dot(m3, W4m_ref[...], preferred_element_type=jnp.float32)
             + ub * w4u_ref[...]
             + b4_ref[...])
        h = jnp.maximum(g, 0.0)
        out = jnp.dot(h, W5_ref[...], preferred_element_type=jnp.float32)
        out_ref[...] = out + b5_ref[...]

    return pl.pallas_call(
        body,
        grid=(grid,),
        in_specs=[
            pl.BlockSpec((BN, 2), lambda i: (i, 0)),
            pl.BlockSpec((BN, 64), lambda i: (i, 0)),
            pl.BlockSpec((BN, 1), lambda i: (i, 0)),
            pl.BlockSpec((BN, 1), lambda i: (i, 0)),
            pl.BlockSpec((BN, 1), lambda i: (i, 0)),
            pl.BlockSpec((G, 1), lambda i: (0, 0)),
            pl.BlockSpec((64, 64), lambda i: (0, 0)),
            pl.BlockSpec((1, 64), lambda i: (0, 0)),
            pl.BlockSpec((2, 67), lambda i: (0, 0)),
            pl.BlockSpec((64, 67), lambda i: (0, 0)),
            pl.BlockSpec((1, 67), lambda i: (0, 0)),
            pl.BlockSpec((1, 67), lambda i: (0, 0)),
            pl.BlockSpec((67, 2), lambda i: (0, 0)),
            pl.BlockSpec((1, 2), lambda i: (0, 0)),
        ],
        out_specs=pl.BlockSpec((BN, 2), lambda i: (i, 0)),
        out_shape=jax.ShapeDtypeStruct((N, 2), jnp.float32),
    )(x, s80, cnt_a, cnt_b, batch2d, u2d, W3, b3, W4x, W4m, w4u, b4, W5, b5)


def kernel(x, edge_index, edge_attr, u, batch, W1, b1, W2, b2, W3, b3, W4, b4, W5, b5):
    N = x.shape[0]
    src = edge_index[0]
    dest = edge_index[1]

    x16 = jnp.pad(x, ((0, 0), (0, 14)))
    xs, cnt2, _unused_parts = _sc_gather(x16, src, dest)
    h2x = _tc_edge_mlp(xs, edge_attr, W1, b1.reshape(1, -1), W2,
                       b2.reshape(1, -1))
    s64 = _sc_scatter(h2x, dest, jnp.zeros((N, 16), jnp.float32))
    out = _tc_node_mlp(
        x, s64, cnt2[0].reshape(-1, 1), cnt2[1].reshape(-1, 1),
        batch.reshape(-1, 1), u.reshape(-1, 1),
        W3, b3.reshape(1, -1),
        W4[0:2, :], W4[2:66, :], W4[66:67, :], b4.reshape(1, -1),
        W5, b5.reshape(1, -1))
    return out


# speedup vs baseline: 2.7173x; 1.3175x over previous
import functools

import jax
import jax.numpy as jnp
from jax import lax
from jax.experimental import pallas as pl
from jax.experimental.pallas import tpu as pltpu
from jax.experimental.pallas import tpu_sc as plsc

NC, NS = 2, 16
NW = NC * NS
CH = 128
BE = 4000
BN = 4000


KB = 4


def _sc_gather(x, src, dest):
    N, D = x.shape
    E = src.shape[0]
    n_chunks = E // CH
    assert n_chunks * CH == E
    src = src.reshape(n_chunks, CH)
    dest = dest.reshape(n_chunks, CH)
    per_worker = -(-n_chunks // NW)
    n_super = -(-per_worker // KB)
    SCH = KB * CH
    RED = 2000
    assert N % RED == 0 and RED % 16 == 0
    n_pieces = N // RED
    pieces_per_tile = -(-n_pieces // NS)

    mesh = plsc.VectorSubcoreMesh(
        core_axis_name="c", subcore_axis_name="s", num_cores=NC, num_subcores=NS)

    @functools.partial(
        pl.kernel,
        out_type=[jax.ShapeDtypeStruct((E, D), jnp.float32),
                  jax.ShapeDtypeStruct((NC, N), jnp.float32),
                  jax.ShapeDtypeStruct((NC, NS, N), jnp.float32)],
        mesh=mesh,
        scratch_types=[
            pltpu.VMEM((KB, CH), jnp.int32),
            pltpu.VMEM((KB, CH), jnp.int32),
            pltpu.VMEM((SCH, 16), jnp.float32),
            pltpu.VMEM((N,), jnp.float32),
            pltpu.VMEM((RED,), jnp.float32),
            pltpu.VMEM((RED,), jnp.float32),
            pltpu.SemaphoreType.DMA,
        ],
        compiler_params=pltpu.CompilerParams(use_tc_tiling_on_sc=False, needs_layout_passes=False),
    )
    def gather_k(x_hbm, src_hbm, dest_hbm, out_hbm, cnt_hbm, part_hbm,
                 sidx_v, didx_v, rows_v, hist_v, tmp_v, red_v, sem_g):
        cid = lax.axis_index("c")
        sid = lax.axis_index("s")
        wid = sid * NC + cid
        ones = jnp.ones((16,), jnp.float32)
        zeros = jnp.zeros((16,), jnp.float32)

        def zbody(i, _):
            hist_v[pl.ds(i * 16, 16)] = zeros
            return ()
        lax.fori_loop(0, N // 16, zbody, (), unroll=False)

        def sbody(g, _):
            c0 = wid * per_worker + g * KB
            base = c0 * CH
            nch = jnp.maximum(jnp.minimum(
                jnp.minimum(per_worker - g * KB, n_chunks - c0), KB), 0)

            @pl.when(nch == KB)
            def _():
                pltpu.sync_copy(src_hbm.at[pl.ds(c0, KB)], sidx_v)
                pltpu.sync_copy(dest_hbm.at[pl.ds(c0, KB)], didx_v)

            @pl.when((nch > 0) & (nch < KB))
            def _():
                for j in range(KB):
                    @pl.when(j < nch)
                    def _():
                        pltpu.sync_copy(src_hbm.at[pl.ds(c0 + j, 1)],
                                        sidx_v.at[pl.ds(j, 1)])
                        pltpu.sync_copy(dest_hbm.at[pl.ds(c0 + j, 1)],
                                        didx_v.at[pl.ds(j, 1)])

            for j in range(KB):
                @pl.when(j < nch)
                def _():
                    pltpu.async_copy(
                        x_hbm.at[sidx_v.at[j]],
                        rows_v.at[pl.ds(j * CH, CH)], sem_g)

            for j in range(KB):
                @pl.when(j < nch)
                def _():
                    def hbody(k, _):
                        d = didx_v[j, pl.ds(k * 16, 16)]
                        plsc.addupdate_scatter(hist_v, [d], ones)
                        return ()
                    lax.fori_loop(0, CH // 16, hbody, (), unroll=False)

            for j in range(KB):
                @pl.when(j < nch)
                def _():
                    pltpu.make_async_copy(
                        x_hbm.at[sidx_v.at[j]],
                        rows_v.at[pl.ds(j * CH, CH)], sem_g).wait()

            @pl.when(nch == KB)
            def _():
                pltpu.sync_copy(rows_v, out_hbm.at[pl.ds(base, SCH)])

            @pl.when((nch > 0) & (nch < KB))
            def _():
                for j in range(KB):
                    @pl.when(j < nch)
                    def _():
                        pltpu.sync_copy(
                            rows_v.at[pl.ds(j * CH, CH)],
                            out_hbm.at[pl.ds(base + j * CH, CH)])
            return ()

        lax.fori_loop(0, n_super, sbody, (), unroll=False)

        pltpu.sync_copy(hist_v, part_hbm.at[cid, sid])
        plsc.subcore_barrier()
        for pp in range(pieces_per_tile):
            piece = pp * NS + sid

            @pl.when(piece < n_pieces)
            def _():
                p0 = piece * RED

                def rzero(i, _):
                    red_v[pl.ds(i * 16, 16)] = zeros
                    return ()
                lax.fori_loop(0, RED // 16, rzero, (), unroll=False)

                for q in range(NS):
                    pltpu.sync_copy(part_hbm.at[cid, q, pl.ds(p0, RED)], tmp_v)

                    def racc(i, _):
                        red_v[pl.ds(i * 16, 16)] = (red_v[pl.ds(i * 16, 16)]
                                                    + tmp_v[pl.ds(i * 16, 16)])
                        return ()
                    lax.fori_loop(0, RED // 16, racc, (), unroll=False)
                pltpu.sync_copy(red_v, cnt_hbm.at[cid, pl.ds(p0, RED)])

    return gather_k(x, src, dest)


def _tc_edge_mlp(xs, ea, W1, b1, W2, b2):
    E = xs.shape[0]
    grid = E // BE
    assert grid * BE == E

    def body(xs_ref, ea_ref, W1_ref, b1_ref, W2_ref, b2_ref, out_ref):
        xs_b = xs_ref[...]
        z = (xs_b[:, 0:1] * W1_ref[0:1, :]
             + xs_b[:, 1:2] * W1_ref[1:2, :]
             + ea_ref[...] * W1_ref[2:3, :]
             + b1_ref[...])
        h1 = jnp.maximum(z, 0.0)
        h2 = jnp.dot(h1, W2_ref[...], preferred_element_type=jnp.float32)
        out_ref[...] = jnp.maximum(h2 + b2_ref[...], 0.0)

    return pl.pallas_call(
        body,
        grid=(grid,),
        in_specs=[
            pl.BlockSpec((BE, 16), lambda i: (i, 0)),
            pl.BlockSpec((BE, 1), lambda i: (i, 0)),
            pl.BlockSpec((3, 64), lambda i: (0, 0)),
            pl.BlockSpec((1, 64), lambda i: (0, 0)),
            pl.BlockSpec((64, 64), lambda i: (0, 0)),
            pl.BlockSpec((1, 64), lambda i: (0, 0)),
        ],
        out_specs=pl.BlockSpec((BE, 64), lambda i: (i, 0)),
        out_shape=jax.ShapeDtypeStruct((E, 64), jnp.float32),
    )(xs, ea, W1, b1, W2, b2)


def _sc_scatter(h2x, dest, zeros16):
    E, W = h2x.shape
    N = zeros16.shape[0]
    n_stripes = W // 16
    assert n_stripes * 16 == W
    n_chunks = E // CH
    assert n_chunks * CH == E
    dest = dest.reshape(n_chunks, CH)
    per_tile = -(-n_chunks // NS)
    n_super = -(-per_tile // KB)
    SCH = KB * CH
    rows_per_tile = N // NS
    assert rows_per_tile * NS == N
    passes = -(-n_stripes // NC)

    mesh = plsc.VectorSubcoreMesh(
        core_axis_name="c", subcore_axis_name="s", num_cores=NC, num_subcores=NS)

    @functools.partial(
        pl.kernel,
        out_type=jax.ShapeDtypeStruct((N, W), jnp.float32),
        mesh=mesh,
        scratch_types=[
            pltpu.VMEM((KB, CH), jnp.int32),
            pltpu.VMEM((SCH, 16), jnp.float32),
            pltpu.VMEM_SHARED((N, 16), jnp.float32),
            pltpu.SemaphoreType.DMA,
        ],
        compiler_params=pltpu.CompilerParams(use_tc_tiling_on_sc=False, needs_layout_passes=False),
    )
    def scatter_k(h2_hbm, dest_hbm, zeros_hbm, out_hbm, didx_v, dat_v, acc_sh,
                  sem_s):
        cid = lax.axis_index("c")
        sid = lax.axis_index("s")
        row0 = sid * rows_per_tile

        for p in range(passes):
            stripe = p * NC + cid

            @pl.when(stripe < n_stripes)
            def _():
                pltpu.sync_copy(zeros_hbm.at[pl.ds(row0, rows_per_tile)],
                                acc_sh.at[pl.ds(row0, rows_per_tile)])

            plsc.subcore_barrier()

            @pl.when(stripe < n_stripes)
            def _():
                col0 = stripe * 16

                def body(g, _):
                    c0 = sid * per_tile + g * KB
                    nch = jnp.maximum(jnp.minimum(
                        jnp.minimum(per_tile - g * KB, n_chunks - c0), KB), 0)

                    @pl.when(nch == KB)
                    def _():
                        pltpu.sync_copy(dest_hbm.at[pl.ds(c0, KB)], didx_v)
                        pltpu.sync_copy(
                            h2_hbm.at[pl.ds(c0 * CH, SCH), pl.ds(col0, 16)],
                            dat_v)

                    @pl.when((nch > 0) & (nch < KB))
                    def _():
                        for j in range(KB):
                            @pl.when(j < nch)
                            def _():
                                pltpu.sync_copy(dest_hbm.at[pl.ds(c0 + j, 1)],
                                                didx_v.at[pl.ds(j, 1)])
                                pltpu.sync_copy(
                                    h2_hbm.at[pl.ds((c0 + j) * CH, CH),
                                              pl.ds(col0, 16)],
                                    dat_v.at[pl.ds(j * CH, CH)])

                    for j in range(KB):
                        @pl.when(j < nch)
                        def _():
                            pltpu.async_copy(
                                dat_v.at[pl.ds(j * CH, CH)],
                                acc_sh.at[didx_v.at[j]], sem_s, add=True)
                    for j in range(KB):
                        @pl.when(j < nch)
                        def _():
                            pltpu.make_async_copy(
                                dat_v.at[pl.ds(j * CH, CH)],
                                acc_sh.at[didx_v.at[j]], sem_s).wait()
                    return ()

                lax.fori_loop(0, n_super, body, (), unroll=False)

            plsc.subcore_barrier()

            @pl.when(stripe < n_stripes)
            def _():
                col0 = stripe * 16
                pltpu.sync_copy(
                    acc_sh.at[pl.ds(row0, rows_per_tile)],
                    out_hbm.at[pl.ds(row0, rows_per_tile), pl.ds(col0, 16)])

            plsc.subcore_barrier()

    return scatter_k(h2x, dest, zeros16)


def _tc_node_mlp(x, s80, cnt_a, cnt_b, batch2d, u2d, W3, b3, W4x, W4m, w4u,
                 b4, W5, b5):
    N = x.shape[0]
    grid = N // BN
    assert grid * BN == N
    G = u2d.shape[0]

    def body(x_ref, s_ref, ca_ref, cb_ref, b_ref, u_ref, W3_ref, b3_ref,
             W4x_ref, W4m_ref, w4u_ref, b4_ref, W5_ref, b5_ref, out_ref):
        sums = s_ref[...]
        c = ca_ref[...] + cb_ref[...]
        mean = sums / jnp.maximum(c, 1.0)
        m3 = jnp.dot(mean, W3_ref[...], preferred_element_type=jnp.float32)
        m3 = jnp.where(c > 0.5, m3 + b3_ref[...], 0.0)
        bb = b_ref[...]
        oh = (lax.broadcasted_iota(jnp.int32, (BN, G), 1) == bb)
        ub = jnp.dot(oh.astype(jnp.float32), u_ref[...],
                     preferred_element_type=jnp.float32)
        xb = x_ref[...]
        g = (xb[:, 0:1] * W4x_ref[0:1, :]
             + xb[:, 1:2] * W4x_ref[1:2, :]
             + jnp.dot(m3, W4m_ref[...], preferred_element_type=jnp.float32)
             + ub * w4u_ref[...]
             + b4_ref[...])
        h = jnp.maximum(g, 0.0)
        out = jnp.dot(h, W5_ref[...], preferred_element_type=jnp.float32)
        out_ref[...] = out + b5_ref[...]

    return pl.pallas_call(
        body,
        grid=(grid,),
        in_specs=[
            pl.BlockSpec((BN, 2), lambda i: (i, 0)),
            pl.BlockSpec((BN, 64), lambda i: (i, 0)),
            pl.BlockSpec((BN, 1), lambda i: (i, 0)),
            pl.BlockSpec((BN, 1), lambda i: (i, 0)),
            pl.BlockSpec((BN, 1), lambda i: (i, 0)),
            pl.BlockSpec((G, 1), lambda i: (0, 0)),
            pl.BlockSpec((64, 64), lambda i: (0, 0)),
            pl.BlockSpec((1, 64), lambda i: (0, 0)),
            pl.BlockSpec((2, 67), lambda i: (0, 0)),
            pl.BlockSpec((64, 67), lambda i: (0, 0)),
            pl.BlockSpec((1, 67), lambda i: (0, 0)),
            pl.BlockSpec((1, 67), lambda i: (0, 0)),
            pl.BlockSpec((67, 2), lambda i: (0, 0)),
            pl.BlockSpec((1, 2), lambda i: (0, 0)),
        ],
        out_specs=pl.BlockSpec((BN, 2), lambda i: (i, 0)),
        out_shape=jax.ShapeDtypeStruct((N, 2), jnp.float32),
    )(x, s80, cnt_a, cnt_b, batch2d, u2d, W3, b3, W4x, W4m, w4u, b4, W5, b5)


def kernel(x, edge_index, edge_attr, u, batch, W1, b1, W2, b2, W3, b3, W4, b4, W5, b5):
    N = x.shape[0]
    src = edge_index[0]
    dest = edge_index[1]

    x16 = jnp.pad(x, ((0, 0), (0, 14)))
    xs, cnt2, _unused_parts = _sc_gather(x16, src, dest)
    h2x = _tc_edge_mlp(xs, edge_attr, W1, b1.reshape(1, -1), W2,
                       b2.reshape(1, -1))
    s64 = _sc_scatter(h2x, dest, jnp.zeros((N, 16), jnp.float32))
    out = _tc_node_mlp(
        x, s64, cnt2[0].reshape(-1, 1), cnt2[1].reshape(-1, 1),
        batch.reshape(-1, 1), u.reshape(-1, 1),
        W3, b3.reshape(1, -1),
        W4[0:2, :], W4[2:66, :], W4[66:67, :], b4.reshape(1, -1),
        W5, b5.reshape(1, -1))
    return out


# speedup vs baseline: 2.9623x; 1.0902x over previous
import functools

import jax
import jax.numpy as jnp
from jax import lax
from jax.experimental import pallas as pl
from jax.experimental.pallas import tpu as pltpu
from jax.experimental.pallas import tpu_sc as plsc

NC, NS = 2, 16
NW = NC * NS
CH = 128
BE = 4000
BN = 4000


KB = 8


def _sc_gather(x, src, dest):
    N, D = x.shape
    E = src.shape[0]
    n_chunks = E // CH
    assert n_chunks * CH == E
    src = src.reshape(n_chunks, CH)
    dest = dest.reshape(n_chunks, CH)
    per_worker = -(-n_chunks // NW)
    n_super = -(-per_worker // KB)
    SCH = KB * CH
    RED = 2000
    assert N % RED == 0 and RED % 16 == 0
    n_pieces = N // RED
    pieces_per_tile = -(-n_pieces // NS)

    mesh = plsc.VectorSubcoreMesh(
        core_axis_name="c", subcore_axis_name="s", num_cores=NC, num_subcores=NS)

    @functools.partial(
        pl.kernel,
        out_type=[jax.ShapeDtypeStruct((E, D), jnp.float32),
                  jax.ShapeDtypeStruct((NC, N), jnp.float32),
                  jax.ShapeDtypeStruct((NC, NS, N), jnp.float32)],
        mesh=mesh,
        scratch_types=[
            pltpu.VMEM((KB, CH), jnp.int32),
            pltpu.VMEM((KB, CH), jnp.int32),
            pltpu.VMEM((SCH, 16), jnp.float32),
            pltpu.VMEM((N,), jnp.float32),
            pltpu.VMEM((RED,), jnp.float32),
            pltpu.VMEM((RED,), jnp.float32),
            pltpu.SemaphoreType.DMA,
        ],
        compiler_params=pltpu.CompilerParams(use_tc_tiling_on_sc=False, needs_layout_passes=False),
    )
    def gather_k(x_hbm, src_hbm, dest_hbm, out_hbm, cnt_hbm, part_hbm,
                 sidx_v, didx_v, rows_v, hist_v, tmp_v, red_v, sem_g):
        cid = lax.axis_index("c")
        sid = lax.axis_index("s")
        wid = sid * NC + cid
        ones = jnp.ones((16,), jnp.float32)
        zeros = jnp.zeros((16,), jnp.float32)

        def zbody(i, _):
            hist_v[pl.ds(i * 16, 16)] = zeros
            return ()
        lax.fori_loop(0, N // 16, zbody, (), unroll=False)

        def sbody(g, _):
            c0 = wid * per_worker + g * KB
            base = c0 * CH
            nch = jnp.maximum(jnp.minimum(
                jnp.minimum(per_worker - g * KB, n_chunks - c0), KB), 0)

            @pl.when(nch == KB)
            def _():
                pltpu.sync_copy(src_hbm.at[pl.ds(c0, KB)], sidx_v)
                pltpu.sync_copy(dest_hbm.at[pl.ds(c0, KB)], didx_v)

            @pl.when((nch > 0) & (nch < KB))
            def _():
                for j in range(KB):
                    @pl.when(j < nch)
                    def _():
                        pltpu.sync_copy(src_hbm.at[pl.ds(c0 + j, 1)],
                                        sidx_v.at[pl.ds(j, 1)])
                        pltpu.sync_copy(dest_hbm.at[pl.ds(c0 + j, 1)],
                                        didx_v.at[pl.ds(j, 1)])

            for j in range(KB):
                @pl.when(j < nch)
                def _():
                    pltpu.async_copy(
                        x_hbm.at[sidx_v.at[j]],
                        rows_v.at[pl.ds(j * CH, CH)], sem_g)

            for j in range(KB):
                @pl.when(j < nch)
                def _():
                    def hbody(k, _):
                        d = didx_v[j, pl.ds(k * 16, 16)]
                        plsc.addupdate_scatter(hist_v, [d], ones)
                        return ()
                    lax.fori_loop(0, CH // 16, hbody, (), unroll=False)

            for j in range(KB):
                @pl.when(j < nch)
                def _():
                    pltpu.make_async_copy(
                        x_hbm.at[sidx_v.at[j]],
                        rows_v.at[pl.ds(j * CH, CH)], sem_g).wait()

            @pl.when(nch == KB)
            def _():
                pltpu.sync_copy(rows_v, out_hbm.at[pl.ds(base, SCH)])

            @pl.when((nch > 0) & (nch < KB))
            def _():
                for j in range(KB):
                    @pl.when(j < nch)
                    def _():
                        pltpu.sync_copy(
                            rows_v.at[pl.ds(j * CH, CH)],
                            out_hbm.at[pl.ds(base + j * CH, CH)])
            return ()

        lax.fori_loop(0, n_super, sbody, (), unroll=False)

        pltpu.sync_copy(hist_v, part_hbm.at[cid, sid])
        plsc.subcore_barrier()
        for pp in range(pieces_per_tile):
            piece = pp * NS + sid

            @pl.when(piece < n_pieces)
            def _():
                p0 = piece * RED

                def rzero(i, _):
                    red_v[pl.ds(i * 16, 16)] = zeros
                    return ()
                lax.fori_loop(0, RED // 16, rzero, (), unroll=False)

                for q in range(NS):
                    pltpu.sync_copy(part_hbm.at[cid, q, pl.ds(p0, RED)], tmp_v)

                    def racc(i, _):
                        red_v[pl.ds(i * 16, 16)] = (red_v[pl.ds(i * 16, 16)]
                                                    + tmp_v[pl.ds(i * 16, 16)])
                        return ()
                    lax.fori_loop(0, RED // 16, racc, (), unroll=False)
                pltpu.sync_copy(red_v, cnt_hbm.at[cid, pl.ds(p0, RED)])

    return gather_k(x, src, dest)


def _tc_edge_mlp(xs, ea, W1, b1, W2, b2):
    E = xs.shape[0]
    grid = E // BE
    assert grid * BE == E

    def body(xs_ref, ea_ref, W1_ref, b1_ref, W2_ref, b2_ref, out_ref):
        xs_b = xs_ref[...]
        z = (xs_b[:, 0:1] * W1_ref[0:1, :]
             + xs_b[:, 1:2] * W1_ref[1:2, :]
             + ea_ref[...] * W1_ref[2:3, :]
             + b1_ref[...])
        h1 = jnp.maximum(z, 0.0)
        h2 = jnp.dot(h1, W2_ref[...], preferred_element_type=jnp.float32)
        out_ref[...] = jnp.maximum(h2 + b2_ref[...], 0.0)

    return pl.pallas_call(
        body,
        grid=(grid,),
        in_specs=[
            pl.BlockSpec((BE, 16), lambda i: (i, 0)),
            pl.BlockSpec((BE, 1), lambda i: (i, 0)),
            pl.BlockSpec((3, 64), lambda i: (0, 0)),
            pl.BlockSpec((1, 64), lambda i: (0, 0)),
            pl.BlockSpec((64, 64), lambda i: (0, 0)),
            pl.BlockSpec((1, 64), lambda i: (0, 0)),
        ],
        out_specs=pl.BlockSpec((BE, 64), lambda i: (i, 0)),
        out_shape=jax.ShapeDtypeStruct((E, 64), jnp.float32),
    )(xs, ea, W1, b1, W2, b2)


def _sc_scatter(h2x, dest, zeros16):
    E, W = h2x.shape
    N = zeros16.shape[0]
    n_stripes = W // 16
    assert n_stripes * 16 == W
    n_chunks = E // CH
    assert n_chunks * CH == E
    dest = dest.reshape(n_chunks, CH)
    per_tile = -(-n_chunks // NS)
    n_super = -(-per_tile // KB)
    SCH = KB * CH
    rows_per_tile = N // NS
    assert rows_per_tile * NS == N
    passes = -(-n_stripes // NC)

    mesh = plsc.VectorSubcoreMesh(
        core_axis_name="c", subcore_axis_name="s", num_cores=NC, num_subcores=NS)

    @functools.partial(
        pl.kernel,
        out_type=jax.ShapeDtypeStruct((N, W), jnp.float32),
        mesh=mesh,
        scratch_types=[
            pltpu.VMEM((KB, CH), jnp.int32),
            pltpu.VMEM((SCH, 16), jnp.float32),
            pltpu.VMEM_SHARED((N, 16), jnp.float32),
            pltpu.SemaphoreType.DMA,
        ],
        compiler_params=pltpu.CompilerParams(use_tc_tiling_on_sc=False, needs_layout_passes=False),
    )
    def scatter_k(h2_hbm, dest_hbm, zeros_hbm, out_hbm, didx_v, dat_v, acc_sh,
                  sem_s):
        cid = lax.axis_index("c")
        sid = lax.axis_index("s")
        row0 = sid * rows_per_tile

        for p in range(passes):
            stripe = p * NC + cid

            @pl.when(stripe < n_stripes)
            def _():
                pltpu.sync_copy(zeros_hbm.at[pl.ds(row0, rows_per_tile)],
                                acc_sh.at[pl.ds(row0, rows_per_tile)])

            plsc.subcore_barrier()

            @pl.when(stripe < n_stripes)
            def _():
                col0 = stripe * 16

                def body(g, _):
                    c0 = sid * per_tile + g * KB
                    nch = jnp.maximum(jnp.minimum(
                        jnp.minimum(per_tile - g * KB, n_chunks - c0), KB), 0)

                    @pl.when(nch == KB)
                    def _():
                        pltpu.sync_copy(dest_hbm.at[pl.ds(c0, KB)], didx_v)
                        pltpu.sync_copy(
                            h2_hbm.at[pl.ds(c0 * CH, SCH), pl.ds(col0, 16)],
                            dat_v)

                    @pl.when((nch > 0) & (nch < KB))
                    def _():
                        for j in range(KB):
                            @pl.when(j < nch)
                            def _():
                                pltpu.sync_copy(dest_hbm.at[pl.ds(c0 + j, 1)],
                                                didx_v.at[pl.ds(j, 1)])
                                pltpu.sync_copy(
                                    h2_hbm.at[pl.ds((c0 + j) * CH, CH),
                                              pl.ds(col0, 16)],
                                    dat_v.at[pl.ds(j * CH, CH)])

                    for j in range(KB):
                        @pl.when(j < nch)
                        def _():
                            pltpu.async_copy(
                                dat_v.at[pl.ds(j * CH, CH)],
                                acc_sh.at[didx_v.at[j]], sem_s, add=True)
                    for j in range(KB):
                        @pl.when(j < nch)
                        def _():
                            pltpu.make_async_copy(
                                dat_v.at[pl.ds(j * CH, CH)],
                                acc_sh.at[didx_v.at[j]], sem_s).wait()
                    return ()

                lax.fori_loop(0, n_super, body, (), unroll=False)

            plsc.subcore_barrier()

            @pl.when(stripe < n_stripes)
            def _():
                col0 = stripe * 16
                pltpu.sync_copy(
                    acc_sh.at[pl.ds(row0, rows_per_tile)],
                    out_hbm.at[pl.ds(row0, rows_per_tile), pl.ds(col0, 16)])

            plsc.subcore_barrier()

    return scatter_k(h2x, dest, zeros16)


def _tc_node_mlp(x, s80, cnt_a, cnt_b, batch2d, u2d, W3, b3, W4x, W4m, w4u,
                 b4, W5, b5):
    N = x.shape[0]
    grid = N // BN
    assert grid * BN == N
    G = u2d.shape[0]

    def body(x_ref, s_ref, ca_ref, cb_ref, b_ref, u_ref, W3_ref, b3_ref,
             W4x_ref, W4m_ref, w4u_ref, b4_ref, W5_ref, b5_ref, out_ref):
        sums = s_ref[...]
        c = ca_ref[...] + cb_ref[...]
        mean = sums / jnp.maximum(c, 1.0)
        m3 = jnp.dot(mean, W3_ref[...], preferred_element_type=jnp.float32)
        m3 = jnp.where(c > 0.5, m3 + b3_ref[...], 0.0)
        bb = b_ref[...]
        oh = (lax.broadcasted_iota(jnp.int32, (BN, G), 1) == bb)
        ub = jnp.dot(oh.astype(jnp.float32), u_ref[...],
                     preferred_element_type=jnp.float32)
        xb = x_ref[...]
        g = (xb[:, 0:1] * W4x_ref[0:1, :]
             + xb[:, 1:2] * W4x_ref[1:2, :]
             + jnp.dot(m3, W4m_ref[...], preferred_element_type=jnp.float32)
             + ub * w4u_ref[...]
             + b4_ref[...])
        h = jnp.maximum(g, 0.0)
        out = jnp.dot(h, W5_ref[...], preferred_element_type=jnp.float32)
        out_ref[...] = out + b5_ref[...]

    return pl.pallas_call(
        body,
        grid=(grid,),
        in_specs=[
            pl.BlockSpec((BN, 2), lambda i: (i, 0)),
            pl.BlockSpec((BN, 64), lambda i: (i, 0)),
            pl.BlockSpec((BN, 1), lambda i: (i, 0)),
            pl.BlockSpec((BN, 1), lambda i: (i, 0)),
            pl.BlockSpec((BN, 1), lambda i: (i, 0)),
            pl.BlockSpec((G, 1), lambda i: (0, 0)),
            pl.BlockSpec((64, 64), lambda i: (0, 0)),
            pl.BlockSpec((1, 64), lambda i: (0, 0)),
            pl.BlockSpec((2, 67), lambda i: (0, 0)),
            pl.BlockSpec((64, 67), lambda i: (0, 0)),
            pl.BlockSpec((1, 67), lambda i: (0, 0)),
            pl.BlockSpec((1, 67), lambda i: (0, 0)),
            pl.BlockSpec((67, 2), lambda i: (0, 0)),
            pl.BlockSpec((1, 2), lambda i: (0, 0)),
        ],
        out_specs=pl.BlockSpec((BN, 2), lambda i: (i, 0)),
        out_shape=jax.ShapeDtypeStruct((N, 2), jnp.float32),
    )(x, s80, cnt_a, cnt_b, batch2d, u2d, W3, b3, W4x, W4m, w4u, b4, W5, b5)


def kernel(x, edge_index, edge_attr, u, batch, W1, b1, W2, b2, W3, b3, W4, b4, W5, b5):
    N = x.shape[0]
    src = edge_index[0]
    dest = edge_index[1]

    x16 = jnp.pad(x, ((0, 0), (0, 14)))
    xs, cnt2, _unused_parts = _sc_gather(x16, src, dest)
    h2x = _tc_edge_mlp(xs, edge_attr, W1, b1.reshape(1, -1), W2,
                       b2.reshape(1, -1))
    s64 = _sc_scatter(h2x, dest, jnp.zeros((N, 16), jnp.float32))
    out = _tc_node_mlp(
        x, s64, cnt2[0].reshape(-1, 1), cnt2[1].reshape(-1, 1),
        batch.reshape(-1, 1), u.reshape(-1, 1),
        W3, b3.reshape(1, -1),
        W4[0:2, :], W4[2:66, :], W4[66:67, :], b4.reshape(1, -1),
        W5, b5.reshape(1, -1))
    return out
